# Initial kernel scaffold; baseline (speedup 1.0000x reference)
import jax
import jax.numpy as jnp
from jax.experimental import pallas as pl


def kernel(movie_id, user_id, user_gender, bucketized_user_age, user_zip_code, user_occupation_text, emb_movie_id, emb_user_id, emb_user_gender, emb_bucketized_user_age, emb_user_zip_code, emb_user_occupation_text, W0, b0, W1, b1, W2, b2, Wout, bout):
    raise NotImplementedError("write your pallas kernel here")


# speedup vs baseline: 1.1274x; 1.1274x over previous
import functools

import jax
import jax.numpy as jnp
from jax import lax
from jax.experimental import pallas as pl
from jax.experimental.pallas import tpu as pltpu
from jax.experimental.pallas import tpu_sc as plsc

B = 16384
EMB = 8
NFEAT = 6
LANES = 128
IDX_CHUNK = 128


def _gather_stage(idx_all, tables):
    info = plsc.get_sparse_core_info()
    nw = info.num_cores * info.num_subcores
    bpw = B // nw
    nchunk = bpw // IDX_CHUNK
    mesh = plsc.VectorSubcoreMesh(core_axis_name="c", subcore_axis_name="s")

    @functools.partial(
        pl.kernel,
        out_type=jax.ShapeDtypeStruct((NFEAT, B, EMB), jnp.float32),
        mesh=mesh,
        scratch_types=[
            pltpu.VMEM((NFEAT, nchunk, IDX_CHUNK), jnp.int32),
            pltpu.VMEM((bpw, EMB), jnp.float32),
            pltpu.SemaphoreType.DMA,
        ],
        compiler_params=pltpu.CompilerParams(use_tc_tiling_on_sc=False),
    )
    def gather_kernel(idx_hbm, t0, t1, t2, t3, t4, t5, out_hbm,
                      idx_v, rows_v, sem):
        tabs = (t0, t1, t2, t3, t4, t5)
        wid = lax.axis_index("s") * info.num_cores + lax.axis_index("c")
        base = wid * bpw
        for f in range(NFEAT):
            for j in range(nchunk):
                pltpu.sync_copy(
                    idx_hbm.at[f, pl.ds(base + j * IDX_CHUNK, IDX_CHUNK)],
                    idx_v.at[f, j],
                )
        for f in range(NFEAT):
            copies = [
                pltpu.async_copy(
                    tabs[f].at[idx_v.at[f, j]],
                    rows_v.at[pl.ds(j * IDX_CHUNK, IDX_CHUNK)],
                    sem,
                )
                for j in range(nchunk)
            ]
            for c in copies:
                c.wait()
            pltpu.sync_copy(rows_v, out_hbm.at[f, pl.ds(base, bpw)])

    return gather_kernel(idx_all, *tables)


def _mlp_kernel(e_ref, w0_ref, b0_ref, w1_ref, b1_ref, w2_ref, b2_ref,
                wo_ref, bo_ref, out_ref):
    w0 = w0_ref[...]
    acc = jnp.zeros(out_ref.shape[:1] + (192,), jnp.float32) + b0_ref[...]
    for f in range(NFEAT):
        acc += jnp.dot(e_ref[f], w0[f * EMB:(f + 1) * EMB, :],
                       preferred_element_type=jnp.float32)
    h = jnp.maximum(acc, 0.0)
    h = jnp.maximum(jnp.dot(h, w1_ref[...],
                            preferred_element_type=jnp.float32)
                    + b1_ref[...], 0.0)
    h = jnp.maximum(jnp.dot(h, w2_ref[...],
                            preferred_element_type=jnp.float32)
                    + b2_ref[...], 0.0)
    out_ref[...] = (jnp.dot(h, wo_ref[...],
                            preferred_element_type=jnp.float32)
                    + bo_ref[...])


def _mlp_stage(egather, W0, b0, W1, b1, W2, b2, Wout, bout):
    bt = 1024
    grid = (B // bt,)
    full2 = lambda shape: pl.BlockSpec(shape, lambda i: (0, 0))
    return pl.pallas_call(
        _mlp_kernel,
        grid=grid,
        in_specs=[
            pl.BlockSpec((NFEAT, bt, EMB), lambda i: (0, i, 0)),
            full2(W0.shape), full2((1, 192)),
            full2(W1.shape), full2((1, 192)),
            full2(W2.shape), full2((1, 192)),
            full2(Wout.shape), full2((1, 1)),
        ],
        out_specs=pl.BlockSpec((bt, 1), lambda i: (i, 0)),
        out_shape=jax.ShapeDtypeStruct((B, 1), jnp.float32),
    )(egather, W0, b0.reshape(1, 192), W1, b1.reshape(1, 192),
      W2, b2.reshape(1, 192), Wout, bout.reshape(1, 1))


def kernel(movie_id, user_id, user_gender, bucketized_user_age,
           user_zip_code, user_occupation_text,
           emb_movie_id, emb_user_id, emb_user_gender,
           emb_bucketized_user_age, emb_user_zip_code,
           emb_user_occupation_text,
           W0, b0, W1, b1, W2, b2, Wout, bout):
    idx_all = jnp.stack([
        movie_id, user_id, user_gender, bucketized_user_age,
        user_zip_code, user_occupation_text,
    ]).astype(jnp.int32)
    tables = (emb_movie_id, emb_user_id, emb_user_gender,
              emb_bucketized_user_age, emb_user_zip_code,
              emb_user_occupation_text)
    egather = _gather_stage(idx_all, tables)
    return _mlp_stage(egather, W0, b0, W1, b1, W2, b2, Wout, bout)


# speedup vs baseline: 1.2782x; 1.1337x over previous
import functools

import jax
import jax.numpy as jnp
from jax import lax
from jax.experimental import pallas as pl
from jax.experimental.pallas import tpu as pltpu
from jax.experimental.pallas import tpu_sc as plsc

B = 16384
EMB = 8
NFEAT = 6
XW = 128
IDX_CHUNK = 128


def _gather_stage(idx_all, tables):
    info = plsc.get_sparse_core_info()
    nw = info.num_cores * info.num_subcores
    bpw = B // nw
    nchunk = bpw // IDX_CHUNK
    mesh = plsc.VectorSubcoreMesh(core_axis_name="c", subcore_axis_name="s")

    @functools.partial(
        pl.kernel,
        out_type=jax.ShapeDtypeStruct((B, XW), jnp.float32),
        mesh=mesh,
        scratch_types=[
            pltpu.VMEM((NFEAT, nchunk, IDX_CHUNK), jnp.int32),
            pltpu.VMEM((NFEAT, bpw, EMB), jnp.float32),
            pltpu.SemaphoreType.DMA,
        ],
        compiler_params=pltpu.CompilerParams(use_tc_tiling_on_sc=False),
    )
    def gather_kernel(idx_hbm, t0, t1, t2, t3, t4, t5, out_hbm,
                      idx_v, rows_v, sem):
        tabs = (t0, t1, t2, t3, t4, t5)
        wid = lax.axis_index("s") * info.num_cores + lax.axis_index("c")
        base = wid * bpw
        for f in range(NFEAT):
            for j in range(nchunk):
                pltpu.sync_copy(
                    idx_hbm.at[f, pl.ds(base + j * IDX_CHUNK, IDX_CHUNK)],
                    idx_v.at[f, j],
                )
        copies = [
            pltpu.async_copy(
                tabs[f].at[idx_v.at[f, j]],
                rows_v.at[f, pl.ds(j * IDX_CHUNK, IDX_CHUNK)],
                sem,
            )
            for f in range(NFEAT)
            for j in range(nchunk)
        ]
        for c in copies:
            c.wait()
        for f in range(NFEAT):
            pltpu.sync_copy(rows_v.at[f],
                            out_hbm.at[pl.ds(base, bpw), pl.ds(f * EMB, EMB)])

    return gather_kernel(idx_all, *tables)


def _mlp_kernel(x_ref, w0_ref, b0_ref, w1_ref, b1_ref, w2_ref, b2_ref,
                wo_ref, bo_ref, out_ref):
    x = x_ref[...][:, :NFEAT * EMB]
    h = jnp.maximum(jnp.dot(x, w0_ref[...],
                            preferred_element_type=jnp.float32)
                    + b0_ref[...], 0.0)
    h = jnp.maximum(jnp.dot(h, w1_ref[...],
                            preferred_element_type=jnp.float32)
                    + b1_ref[...], 0.0)
    h = jnp.maximum(jnp.dot(h, w2_ref[...],
                            preferred_element_type=jnp.float32)
                    + b2_ref[...], 0.0)
    out_ref[...] = (jnp.dot(h, wo_ref[...],
                            preferred_element_type=jnp.float32)
                    + bo_ref[...])


def _mlp_stage(xcat, W0, b0, W1, b1, W2, b2, Wout, bout):
    bt = 2048
    grid = (B // bt,)
    full2 = lambda shape: pl.BlockSpec(shape, lambda i: (0, 0))
    return pl.pallas_call(
        _mlp_kernel,
        grid=grid,
        in_specs=[
            pl.BlockSpec((bt, XW), lambda i: (i, 0)),
            full2(W0.shape), full2((1, 192)),
            full2(W1.shape), full2((1, 192)),
            full2(W2.shape), full2((1, 192)),
            full2(Wout.shape), full2((1, 1)),
        ],
        out_specs=pl.BlockSpec((bt, 1), lambda i: (i, 0)),
        out_shape=jax.ShapeDtypeStruct((B, 1), jnp.float32),
    )(xcat, W0, b0.reshape(1, 192), W1, b1.reshape(1, 192),
      W2, b2.reshape(1, 192), Wout, bout.reshape(1, 1))


def kernel(movie_id, user_id, user_gender, bucketized_user_age,
           user_zip_code, user_occupation_text,
           emb_movie_id, emb_user_id, emb_user_gender,
           emb_bucketized_user_age, emb_user_zip_code,
           emb_user_occupation_text,
           W0, b0, W1, b1, W2, b2, Wout, bout):
    idx_all = jnp.stack([
        movie_id, user_id, user_gender, bucketized_user_age,
        user_zip_code, user_occupation_text,
    ]).astype(jnp.int32)
    tables = (emb_movie_id, emb_user_id, emb_user_gender,
              emb_bucketized_user_age, emb_user_zip_code,
              emb_user_occupation_text)
    xcat = _gather_stage(idx_all, tables)
    return _mlp_stage(xcat, W0, b0, W1, b1, W2, b2, Wout, bout)


# speedup vs baseline: 1.5401x; 1.2049x over previous
import functools

import jax
import jax.numpy as jnp
from jax import lax
from jax.experimental import pallas as pl
from jax.experimental.pallas import tpu as pltpu
from jax.experimental.pallas import tpu_sc as plsc

B = 16384
EMB = 8
NFEAT = 6
V = 100001
VPAD = ((V + 15) // 16) * 16
XW = 128
IDX_CHUNK = 128
RB = 512


def _repack_kernel(*refs):
    in_refs, out_refs = refs[:NFEAT], refs[NFEAT:]
    for x_ref, o_ref in zip(in_refs, out_refs):
        x = x_ref[...]
        o_ref[...] = x.reshape(8, RB, 16).transpose(1, 2, 0).reshape(RB, 128)


def _repack_stage(tables):
    nrow = VPAD // 16
    grid = ((nrow + RB - 1) // RB,)
    t8s = [t.T for t in tables]
    out = pl.pallas_call(
        _repack_kernel,
        grid=grid,
        in_specs=[pl.BlockSpec((8, RB * 16), lambda i: (0, i))
                  for _ in range(NFEAT)],
        out_specs=[pl.BlockSpec((RB, 128), lambda i: (i, 0))
                   for _ in range(NFEAT)],
        out_shape=[jax.ShapeDtypeStruct((nrow, 128), jnp.float32)
                   for _ in range(NFEAT)],
    )(*t8s)
    return [p.reshape(VPAD, EMB) for p in out]


def _gather_stage(idx_all, tables):
    info = plsc.get_sparse_core_info()
    nw = info.num_cores * info.num_subcores
    bpw = B // nw
    nchunk = bpw // IDX_CHUNK
    mesh = plsc.VectorSubcoreMesh(core_axis_name="c", subcore_axis_name="s")

    @functools.partial(
        pl.kernel,
        out_type=jax.ShapeDtypeStruct((B, XW), jnp.float32),
        mesh=mesh,
        scratch_types=[
            pltpu.VMEM((NFEAT, nchunk, IDX_CHUNK), jnp.int32),
            pltpu.VMEM((NFEAT, bpw, EMB), jnp.float32),
            pltpu.SemaphoreType.DMA,
        ],
        compiler_params=pltpu.CompilerParams(use_tc_tiling_on_sc=False),
    )
    def gather_kernel(idx_hbm, t0, t1, t2, t3, t4, t5, out_hbm,
                      idx_v, rows_v, sem):
        tabs = (t0, t1, t2, t3, t4, t5)
        wid = lax.axis_index("s") * info.num_cores + lax.axis_index("c")
        base = wid * bpw
        for f in range(NFEAT):
            for j in range(nchunk):
                pltpu.sync_copy(
                    idx_hbm.at[f, pl.ds(base + j * IDX_CHUNK, IDX_CHUNK)],
                    idx_v.at[f, j],
                )
        copies = [
            pltpu.async_copy(
                tabs[f].at[idx_v.at[f, j]],
                rows_v.at[f, pl.ds(j * IDX_CHUNK, IDX_CHUNK)],
                sem,
            )
            for f in range(NFEAT)
            for j in range(nchunk)
        ]
        for c in copies:
            c.wait()
        for f in range(NFEAT):
            pltpu.sync_copy(rows_v.at[f],
                            out_hbm.at[pl.ds(base, bpw), pl.ds(f * EMB, EMB)])

    return gather_kernel(idx_all, *tables)


def _mlp_kernel(x_ref, w0_ref, b0_ref, w1_ref, b1_ref, w2_ref, b2_ref,
                wo_ref, bo_ref, out_ref):
    x = x_ref[...][:, :NFEAT * EMB]
    h = jnp.maximum(jnp.dot(x, w0_ref[...],
                            preferred_element_type=jnp.float32)
                    + b0_ref[...], 0.0)
    h = jnp.maximum(jnp.dot(h, w1_ref[...],
                            preferred_element_type=jnp.float32)
                    + b1_ref[...], 0.0)
    h = jnp.maximum(jnp.dot(h, w2_ref[...],
                            preferred_element_type=jnp.float32)
                    + b2_ref[...], 0.0)
    out_ref[...] = (jnp.dot(h, wo_ref[...],
                            preferred_element_type=jnp.float32)
                    + bo_ref[...])


def _mlp_stage(xcat, W0, b0, W1, b1, W2, b2, Wout, bout):
    bt = 2048
    grid = (B // bt,)
    full2 = lambda shape: pl.BlockSpec(shape, lambda i: (0, 0))
    return pl.pallas_call(
        _mlp_kernel,
        grid=grid,
        in_specs=[
            pl.BlockSpec((bt, XW), lambda i: (i, 0)),
            full2(W0.shape), full2((1, 192)),
            full2(W1.shape), full2((1, 192)),
            full2(W2.shape), full2((1, 192)),
            full2(Wout.shape), full2((1, 1)),
        ],
        out_specs=pl.BlockSpec((bt, 1), lambda i: (i, 0)),
        out_shape=jax.ShapeDtypeStruct((B, 1), jnp.float32),
    )(xcat, W0, b0.reshape(1, 192), W1, b1.reshape(1, 192),
      W2, b2.reshape(1, 192), Wout, bout.reshape(1, 1))


def kernel(movie_id, user_id, user_gender, bucketized_user_age,
           user_zip_code, user_occupation_text,
           emb_movie_id, emb_user_id, emb_user_gender,
           emb_bucketized_user_age, emb_user_zip_code,
           emb_user_occupation_text,
           W0, b0, W1, b1, W2, b2, Wout, bout):
    idx_all = jnp.stack([
        movie_id, user_id, user_gender, bucketized_user_age,
        user_zip_code, user_occupation_text,
    ]).astype(jnp.int32)
    tables = (emb_movie_id, emb_user_id, emb_user_gender,
              emb_bucketized_user_age, emb_user_zip_code,
              emb_user_occupation_text)
    packed = _repack_stage(tables)
    xcat = _gather_stage(idx_all, packed)
    return _mlp_stage(xcat, W0, b0, W1, b1, W2, b2, Wout, bout)


# speedup vs baseline: 1.5859x; 1.0298x over previous
import functools

import jax
import jax.numpy as jnp
from jax import lax
from jax.experimental import pallas as pl
from jax.experimental.pallas import tpu as pltpu
from jax.experimental.pallas import tpu_sc as plsc

B = 16384
EMB = 8
NFEAT = 6
V = 100001
VPAD = ((V + 15) // 16) * 16
XW = 128
IDX_CHUNK = 128
RB = 512


def _repack_kernel(*refs):
    in_refs, out_refs = refs[:NFEAT], refs[NFEAT:]
    for x_ref, o_ref in zip(in_refs, out_refs):
        x = x_ref[...]
        o_ref[...] = x.reshape(8, RB, 16).transpose(1, 2, 0).reshape(RB, 128)


def _repack_stage(tables):
    nrow = VPAD // 16
    grid = ((nrow + RB - 1) // RB,)
    t8s = [t.T for t in tables]
    out = pl.pallas_call(
        _repack_kernel,
        grid=grid,
        in_specs=[pl.BlockSpec((8, RB * 16), lambda i: (0, i))
                  for _ in range(NFEAT)],
        out_specs=[pl.BlockSpec((RB, 128), lambda i: (i, 0))
                   for _ in range(NFEAT)],
        out_shape=[jax.ShapeDtypeStruct((nrow, 128), jnp.float32)
                   for _ in range(NFEAT)],
    )(*t8s)
    return [p.reshape(VPAD, EMB) for p in out]


def _gather_stage(idx_all, tables):
    info = plsc.get_sparse_core_info()
    nw = info.num_cores * info.num_subcores
    bpw = B // nw
    nchunk = bpw // IDX_CHUNK
    mesh = plsc.VectorSubcoreMesh(core_axis_name="c", subcore_axis_name="s")

    @functools.partial(
        pl.kernel,
        out_type=jax.ShapeDtypeStruct((B, XW), jnp.float32),
        mesh=mesh,
        scratch_types=[
            pltpu.VMEM((NFEAT, bpw), jnp.int32),
            pltpu.VMEM((NFEAT, bpw, EMB), jnp.float32),
            pltpu.SemaphoreType.DMA,
        ],
        compiler_params=pltpu.CompilerParams(use_tc_tiling_on_sc=False),
    )
    def gather_kernel(idx_hbm, t0, t1, t2, t3, t4, t5, out_hbm,
                      idx_v, rows_v, sem):
        tabs = (t0, t1, t2, t3, t4, t5)
        wid = lax.axis_index("s") * info.num_cores + lax.axis_index("c")
        base = wid * bpw
        for f in range(NFEAT):
            pltpu.sync_copy(idx_hbm.at[f, pl.ds(base, bpw)], idx_v.at[f])
        copies = [
            pltpu.async_copy(
                tabs[f].at[idx_v.at[f, pl.ds(j * IDX_CHUNK, IDX_CHUNK)]],
                rows_v.at[f, pl.ds(j * IDX_CHUNK, IDX_CHUNK)],
                sem,
            )
            for f in range(NFEAT)
            for j in range(nchunk)
        ]
        for c in copies:
            c.wait()
        for f in range(NFEAT):
            pltpu.sync_copy(rows_v.at[f],
                            out_hbm.at[pl.ds(base, bpw), pl.ds(f * EMB, EMB)])

    return gather_kernel(idx_all, *tables)


def _mlp_kernel(x_ref, w0_ref, b0_ref, w1_ref, b1_ref, w2_ref, b2_ref,
                wo_ref, bo_ref, out_ref):
    x = x_ref[...][:, :NFEAT * EMB]
    h = jnp.maximum(jnp.dot(x, w0_ref[...],
                            preferred_element_type=jnp.float32)
                    + b0_ref[...], 0.0)
    h = jnp.maximum(jnp.dot(h, w1_ref[...],
                            preferred_element_type=jnp.float32)
                    + b1_ref[...], 0.0)
    h = jnp.maximum(jnp.dot(h, w2_ref[...],
                            preferred_element_type=jnp.float32)
                    + b2_ref[...], 0.0)
    out_ref[...] = (jnp.dot(h, wo_ref[...],
                            preferred_element_type=jnp.float32)
                    + bo_ref[...])


def _mlp_stage(xcat, W0, b0, W1, b1, W2, b2, Wout, bout):
    bt = 4096
    grid = (B // bt,)
    full2 = lambda shape: pl.BlockSpec(shape, lambda i: (0, 0))
    return pl.pallas_call(
        _mlp_kernel,
        grid=grid,
        in_specs=[
            pl.BlockSpec((bt, XW), lambda i: (i, 0)),
            full2(W0.shape), full2((1, 192)),
            full2(W1.shape), full2((1, 192)),
            full2(W2.shape), full2((1, 192)),
            full2(Wout.shape), full2((1, 1)),
        ],
        out_specs=pl.BlockSpec((bt, 1), lambda i: (i, 0)),
        out_shape=jax.ShapeDtypeStruct((B, 1), jnp.float32),
    )(xcat, W0, b0.reshape(1, 192), W1, b1.reshape(1, 192),
      W2, b2.reshape(1, 192), Wout, bout.reshape(1, 1))


def kernel(movie_id, user_id, user_gender, bucketized_user_age,
           user_zip_code, user_occupation_text,
           emb_movie_id, emb_user_id, emb_user_gender,
           emb_bucketized_user_age, emb_user_zip_code,
           emb_user_occupation_text,
           W0, b0, W1, b1, W2, b2, Wout, bout):
    idx_all = jnp.stack([
        movie_id, user_id, user_gender, bucketized_user_age,
        user_zip_code, user_occupation_text,
    ]).astype(jnp.int32)
    tables = (emb_movie_id, emb_user_id, emb_user_gender,
              emb_bucketized_user_age, emb_user_zip_code,
              emb_user_occupation_text)
    packed = _repack_stage(tables)
    xcat = _gather_stage(idx_all, packed)
    return _mlp_stage(xcat, W0, b0, W1, b1, W2, b2, Wout, bout)
